# Initial kernel scaffold; baseline (speedup 1.0000x reference)
import jax
import jax.numpy as jnp
from jax.experimental import pallas as pl


def kernel(x, edge_index, y, pair_src, pair_tgt, W_in, b_in, W_ih, W_hh, W1, b1, W2, b2):
    raise NotImplementedError("write your pallas kernel here")


# speedup vs baseline: 4.2149x; 4.2149x over previous
import functools

import jax
import jax.numpy as jnp
from jax import lax
from jax.experimental import pallas as pl
from jax.experimental.pallas import tpu as pltpu
from jax.experimental.pallas import tpu_sc as plsc

N = 10000
E = 320000
NPAIRS = 50000
HID = 128

NW = 32
EPAD = 327680
EPW = EPAD // NW
ECHUNK = 128
NACC = 10112
PPAD = 53248
PPW = PPAD // NW
NBLK = 1000
NGRID = N // NBLK

_f32 = jnp.float32


def _mesh():
    return plsc.VectorSubcoreMesh(core_axis_name="c", subcore_axis_name="s",
                                  num_cores=2, num_subcores=16)


@functools.cache
def _sc_deg_kernel():
    return functools.partial(
        pl.kernel,
        out_type=jax.ShapeDtypeStruct((2, NACC, HID), _f32),
        mesh=_mesh(),
        scratch_types=[
            pltpu.VMEM((4, ECHUNK), jnp.int32),
            pltpu.VMEM((ECHUNK, HID), _f32),
            pltpu.VMEM_SHARED((NACC, HID), _f32),
        ] + [pltpu.SemaphoreType.DMA] * 4,
    )(_sc_deg_body)


def _sc_deg_body(col_hbm, ones_hbm, zeros_hbm, out_hbm, coli_v, ones_v,
                 acc_sh, *sems):
    cid = lax.axis_index("c")
    sid = lax.axis_index("s")
    wid = sid * 2 + cid
    pltpu.sync_copy(ones_hbm, ones_v)
    pltpu.sync_copy(zeros_hbm, acc_sh.at[pl.ds(sid * 632, 632)])
    plsc.subcore_barrier()

    def group(gi, c):
        for b in range(4):
            pltpu.sync_copy(col_hbm.at[wid, gi * 4 + b], coli_v.at[b])
        sd = []
        for b in range(4):
            sd.append(pltpu.async_copy(
                ones_v, acc_sh.at[coli_v.at[b]], sems[b], add=True))
        for b in range(4):
            sd[b].wait()
        return c

    lax.fori_loop(0, EPW // ECHUNK // 4, group, 0)
    plsc.subcore_barrier()
    pltpu.sync_copy(acc_sh.at[pl.ds(sid * 632, 632)],
                    out_hbm.at[cid, pl.ds(sid * 632, 632)])


NBUF = 2
NCH = EPW // ECHUNK
NGRP = NCH // NBUF


@functools.cache
def _sc_agg_kernel():
    return functools.partial(
        pl.kernel,
        out_type=jax.ShapeDtypeStruct((2, NACC, HID), _f32),
        mesh=_mesh(),
        scratch_types=[
            pltpu.VMEM((NCH, ECHUNK), jnp.int32),
            pltpu.VMEM((NBUF, ECHUNK), jnp.int32),
        ] + [pltpu.VMEM((ECHUNK, HID), _f32)] * NBUF
          + [pltpu.VMEM_SHARED((NACC, HID), _f32)]
          + [pltpu.SemaphoreType.DMA] * (2 * NBUF),
    )(_sc_agg_body)


def _sc_agg_body(tbl_hbm, row_hbm, col_hbm, zeros_hbm, out_hbm,
                 row2_v, coli_v, *rest):
    bufs = rest[:NBUF]
    acc_sh = rest[NBUF]
    gsems = rest[NBUF + 1:2 * NBUF + 1]
    ssems = rest[2 * NBUF + 1:]
    cid = lax.axis_index("c")
    sid = lax.axis_index("s")
    wid = sid * 2 + cid
    pltpu.sync_copy(row_hbm.at[wid], row2_v)
    pltpu.sync_copy(zeros_hbm, acc_sh.at[pl.ds(sid * 632, 632)])
    plsc.subcore_barrier()

    def group(gi, c):
        gd = []
        for b in range(NBUF):
            gd.append(pltpu.async_copy(
                tbl_hbm.at[row2_v.at[gi * NBUF + b]], bufs[b], gsems[b]))
        for b in range(NBUF):
            pltpu.sync_copy(col_hbm.at[wid, gi * NBUF + b], coli_v.at[b])
        sd = []
        for b in range(NBUF):
            gd[b].wait()
            sd.append(pltpu.async_copy(
                bufs[b], acc_sh.at[coli_v.at[b]], ssems[b], add=True))
        for b in range(NBUF):
            sd[b].wait()
        return c

    lax.fori_loop(0, NGRP, group, 0)
    plsc.subcore_barrier()
    pltpu.sync_copy(acc_sh.at[pl.ds(sid * 632, 632)],
                    out_hbm.at[cid, pl.ds(sid * 632, 632)])


@functools.cache
def _sc_pairs_kernel():
    return functools.partial(
        pl.kernel,
        out_type=(jax.ShapeDtypeStruct((PPAD * 16,), _f32),
                  jax.ShapeDtypeStruct((PPAD * 16,), _f32)),
        mesh=_mesh(),
        scratch_types=[
            pltpu.VMEM((PPW,), jnp.int32),
            pltpu.VMEM((PPW,), jnp.int32),
            pltpu.VMEM((128, HID), _f32),
            pltpu.VMEM((128, HID), _f32),
            pltpu.VMEM((PPW * 16,), _f32),
            pltpu.VMEM((PPW * 16,), _f32),
            pltpu.SemaphoreType.DMA,
        ],
    )(_sc_pairs_body)


def _sc_pairs_body(py_hbm, src_hbm, tgt_hbm, d_out, dl_out,
                   src_v, tgt_v, s_chunk, t_chunk, d_v, dl_v, sem):
    cid = lax.axis_index("c")
    sid = lax.axis_index("s")
    wid = sid * 2 + cid
    base = wid * PPW
    pltpu.sync_copy(src_hbm.at[pl.ds(base, PPW)], src_v)
    pltpu.sync_copy(tgt_hbm.at[pl.ds(base, PPW)], tgt_v)

    def body(i, c):
        sl = pl.ds(pl.multiple_of(i * 128, 128), 128)
        a = pltpu.async_copy(py_hbm.at[src_v.at[sl]], s_chunk, sem)
        b = pltpu.async_copy(py_hbm.at[tgt_v.at[sl]], t_chunk, sem)
        a.wait()
        b.wait()

        def inner(j, c2):
            r = pl.ds(pl.multiple_of((i * 128 + j) * 16, 16), 16)
            d_v[r] = s_chunk[j, pl.ds(0, 16)] - t_chunk[j, pl.ds(0, 16)]
            dl_v[r] = s_chunk[j, pl.ds(16, 16)] - t_chunk[j, pl.ds(16, 16)]
            return c2

        return lax.fori_loop(0, 128, inner, c)

    lax.fori_loop(0, PPW // 128, body, 0)
    pltpu.sync_copy(d_v, d_out.at[pl.ds(base * 16, PPW * 16)])
    pltpu.sync_copy(dl_v, dl_out.at[pl.ds(base * 16, PPW * 16)])


def _tc_init_body(xp_ref, wint_ref, bin_ref, deg0_ref, deg1_ref,
                  h_ref, g_ref, dis_ref):
    h0 = jnp.maximum(
        jnp.dot(xp_ref[...], wint_ref[...], preferred_element_type=_f32)
        + bin_ref[...], 0.0)
    s = jnp.sum(h0 * h0, axis=1, keepdims=True)
    h0 = h0 * lax.rsqrt(jnp.maximum(s, 1e-24))
    deg = deg0_ref[0] + deg1_ref[0] + 1.0
    dis = lax.rsqrt(deg)
    h_ref[...] = h0
    dis_ref[...] = dis
    g_ref[...] = dis * h0


def _tc_init(xp, wint, bin2, deg):
    return pl.pallas_call(
        _tc_init_body,
        grid=(NGRID,),
        in_specs=[
            pl.BlockSpec((NBLK, HID), lambda i: (i, 0)),
            pl.BlockSpec((HID, HID), lambda i: (0, 0)),
            pl.BlockSpec((1, HID), lambda i: (0, 0)),
            pl.BlockSpec((1, NBLK, HID), lambda i: (0, i, 0)),
            pl.BlockSpec((1, NBLK, HID), lambda i: (1, i, 0)),
        ],
        out_specs=[
            pl.BlockSpec((NBLK, HID), lambda i: (i, 0)),
            pl.BlockSpec((NBLK, HID), lambda i: (i, 0)),
            pl.BlockSpec((NBLK, HID), lambda i: (i, 0)),
        ],
        out_shape=[jax.ShapeDtypeStruct((N, HID), _f32)] * 3,
    )(xp, wint, bin2, deg, deg)


def _tc_gru_body(h_ref, p0_ref, p1_ref, dis_ref, wih_ref, whh_ref, hmax_ref,
                 ho_ref, go_ref, mo_ref):
    dis = dis_ref[...]
    hn = dis * (p0_ref[0] + p1_ref[0])
    h = h_ref[...]
    gi = jnp.dot(h, wih_ref[...], preferred_element_type=_f32)
    gh = jnp.dot(hn, whh_ref[...], preferred_element_type=_f32)
    r = jax.nn.sigmoid(gi[:, :HID] + gh[:, :HID])
    z = jax.nn.sigmoid(gi[:, HID:2 * HID] + gh[:, HID:2 * HID])
    nc = jnp.tanh(gi[:, 2 * HID:] + r * gh[:, 2 * HID:])
    hnew = (1.0 - z) * nc + z * hn
    s = jnp.sum(hnew * hnew, axis=1, keepdims=True)
    hnew = hnew * lax.rsqrt(jnp.maximum(s, 1e-24))
    ho_ref[...] = hnew
    go_ref[...] = dis * hnew
    mo_ref[...] = jnp.maximum(hmax_ref[...], hnew)


def _tc_gru(h, parts, dis2d, wiht, whht, hmax):
    return pl.pallas_call(
        _tc_gru_body,
        grid=(NGRID,),
        in_specs=[
            pl.BlockSpec((NBLK, HID), lambda i: (i, 0)),
            pl.BlockSpec((1, NBLK, HID), lambda i: (0, i, 0)),
            pl.BlockSpec((1, NBLK, HID), lambda i: (1, i, 0)),
            pl.BlockSpec((NBLK, HID), lambda i: (i, 0)),
            pl.BlockSpec((HID, 3 * HID), lambda i: (0, 0)),
            pl.BlockSpec((HID, 3 * HID), lambda i: (0, 0)),
            pl.BlockSpec((NBLK, HID), lambda i: (i, 0)),
        ],
        out_specs=[
            pl.BlockSpec((NBLK, HID), lambda i: (i, 0)),
            pl.BlockSpec((NBLK, HID), lambda i: (i, 0)),
            pl.BlockSpec((NBLK, HID), lambda i: (i, 0)),
        ],
        out_shape=[jax.ShapeDtypeStruct((N, HID), _f32)] * 3,
    )(h, parts, parts, dis2d, wiht, whht, hmax)


def _tc_mlp_body(hmax_ref, w1t_ref, b1_ref, w2_ref, b2_ref, y_ref,
                 out_ref, outpy_ref):
    hm = hmax_ref[...]
    s = jnp.sum(hm * hm, axis=1, keepdims=True)
    z = hm * lax.rsqrt(jnp.maximum(s, 1e-24))
    hid = jnp.maximum(
        jnp.dot(z, w1t_ref[...], preferred_element_type=_f32) + b1_ref[...],
        0.0)
    pred = jnp.sum(hid * w2_ref[...], axis=1) + b2_ref[0, 0]
    pred = jnp.maximum(pred, 0.0)
    out_ref[0, 0, :] = pred
    yb = y_ref[0, 0, :]
    outpy_ref[...] = jnp.concatenate(
        [jnp.broadcast_to(pred[:, None], (NBLK, 16)),
         jnp.broadcast_to(yb[:, None], (NBLK, 16)),
         jnp.zeros((NBLK, HID - 32), _f32)], axis=1)


def _tc_mlp(hmax, w1t, b1p, w2row, b2b, y3):
    return pl.pallas_call(
        _tc_mlp_body,
        grid=(NGRID,),
        in_specs=[
            pl.BlockSpec((NBLK, HID), lambda i: (i, 0)),
            pl.BlockSpec((HID, HID), lambda i: (0, 0)),
            pl.BlockSpec((1, HID), lambda i: (0, 0)),
            pl.BlockSpec((1, HID), lambda i: (0, 0)),
            pl.BlockSpec((1, HID), lambda i: (0, 0)),
            pl.BlockSpec((1, 1, NBLK), lambda i: (i, 0, 0)),
        ],
        out_specs=[
            pl.BlockSpec((1, 1, NBLK), lambda i: (i, 0, 0)),
            pl.BlockSpec((NBLK, HID), lambda i: (i, 0)),
        ],
        out_shape=[jax.ShapeDtypeStruct((NGRID, 1, NBLK), _f32),
                   jax.ShapeDtypeStruct((N, HID), _f32)],
    )(hmax, w1t, b1p, w2row, b2b, y3)


def _tc_loss_body(d_ref, dl_ref, out_ref):
    d = d_ref[...]
    dl = dl_ref[...]
    p = jax.nn.sigmoid(d)
    lab = jax.nn.sigmoid(dl)
    eps = 1e-12
    p = jnp.clip(p, eps, 1.0 - eps)
    rows = lax.broadcasted_iota(jnp.int32, d.shape, 0)
    lanes = lax.broadcasted_iota(jnp.int32, d.shape, 1)
    pair_idx = rows * 8 + lanes // 16
    mask = pair_idx < NPAIRS
    t = lab * jnp.log(p) + (1.0 - lab) * jnp.log(1.0 - p)
    loss = -jnp.sum(jnp.where(mask, t, 0.0)) / (16.0 * NPAIRS)
    out_ref[...] = jnp.full((8, 128), loss, _f32)


def _tc_loss(d, dl):
    return pl.pallas_call(
        _tc_loss_body,
        out_shape=jax.ShapeDtypeStruct((8, 128), _f32),
    )(d, dl)


def kernel(x, edge_index, y, pair_src, pair_tgt,
           W_in, b_in, W_ih, W_hh, W1, b1, W2, b2):
    row = edge_index[0]
    col = edge_index[1]
    rowp = jnp.concatenate([row, jnp.zeros((EPAD - E,), jnp.int32)]
                           ).reshape(NW, EPW // ECHUNK, ECHUNK)
    colp = jnp.concatenate([col, jnp.full((EPAD - E,), N, jnp.int32)]
                           ).reshape(NW, EPW // ECHUNK, ECHUNK)
    srcp = jnp.concatenate([pair_src, jnp.zeros((PPAD - NPAIRS,), jnp.int32)])
    tgtp = jnp.concatenate([pair_tgt, jnp.zeros((PPAD - NPAIRS,), jnp.int32)])

    xp = jnp.pad(x, ((0, 0), (0, HID - x.shape[1])))
    wint = jnp.pad(W_in.T, ((0, HID - x.shape[1]), (0, 0)))
    bin2 = b_in.reshape(1, HID)
    wiht = W_ih.T
    whht = W_hh.T
    w1t = jnp.pad(W1.T, ((0, 0), (0, HID - W1.shape[0])))
    b1p = jnp.pad(b1, (0, HID - b1.shape[0])).reshape(1, HID)
    w2row = jnp.pad(W2[0], (0, HID - W2.shape[1])).reshape(1, HID)
    b2b = jnp.broadcast_to(b2.reshape(1, 1), (1, HID))

    ones128 = jnp.ones((ECHUNK, HID), _f32)
    zerosh = jnp.zeros((632, HID), _f32)

    deg = _sc_deg_kernel()(colp, ones128, zerosh)
    h, g, dis2d = _tc_init(xp, wint, bin2, deg)
    hmax = h
    sc_agg = _sc_agg_kernel()
    for _ in range(5):
        parts = sc_agg(g, rowp, colp, zerosh)
        h, g, hmax = _tc_gru(h, parts, dis2d, wiht, whht, hmax)
    pred3, py128 = _tc_mlp(hmax, w1t, b1p, w2row, b2b,
                           y.reshape(NGRID, 1, NBLK))
    pred = pred3.reshape(N)
    d, dl = _sc_pairs_kernel()(py128, srcp, tgtp)
    lossm = _tc_loss(d.reshape(PPAD * 16 // 128, 128),
                     dl.reshape(PPAD * 16 // 128, 128))
    return pred, lossm[0, 0]
